# Initial kernel scaffold; baseline (speedup 1.0000x reference)
import jax
import jax.numpy as jnp
from jax.experimental import pallas as pl


def kernel(x, table, W1, b1, W2, b2):
    raise NotImplementedError("write your pallas kernel here")


# speedup vs baseline: 3.4337x; 3.4337x over previous
import functools

import jax
import jax.numpy as jnp
from jax import lax
from jax.experimental import pallas as pl
from jax.experimental.pallas import tpu as pltpu
from jax.experimental.pallas import tpu_sc as plsc

EMBED = 64
GATHER_WINDOW = 128
MLP_ROWS = 1024


def _sc_gather(table, idx):
    n_idx = idx.shape[1]
    mesh = plsc.VectorSubcoreMesh(core_axis_name="c", subcore_axis_name="s")

    @functools.partial(
        pl.kernel,
        out_type=jax.ShapeDtypeStruct((n_idx, EMBED), jnp.float32),
        mesh=mesh,
        compiler_params=pltpu.CompilerParams(use_tc_tiling_on_sc=False),
    )
    def gather_kernel(table_hbm, idx_hbm, out_hbm):
        def body(i_vmem, o_vmem):
            pltpu.sync_copy(table_hbm.at[i_vmem.at[0]], o_vmem)

        pltpu.emit_pipeline(
            body,
            grid=(n_idx // GATHER_WINDOW,),
            in_specs=[pl.BlockSpec((1, GATHER_WINDOW), lambda i: (0, i))],
            out_specs=[pl.BlockSpec((GATHER_WINDOW, EMBED), lambda i: (i, 0))],
            core_axis_name=("c", "s"),
            dimension_semantics=(pltpu.PARALLEL,),
        )(idx_hbm, out_hbm)

    return gather_kernel(table, idx)


def _tc_mlp(emb, W1, b1, W2, b2):
    n = emb.shape[0]

    def body(emb_ref, w1_ref, b1_ref, w2_ref, b2_ref, out_ref):
        h = jnp.dot(emb_ref[...], w1_ref[...],
                    preferred_element_type=jnp.float32,
                    precision=lax.Precision.HIGHEST) + b1_ref[...]
        h = h * jax.nn.sigmoid(h)
        out_ref[...] = jnp.dot(h, w2_ref[...],
                               preferred_element_type=jnp.float32,
                               precision=lax.Precision.HIGHEST) + b2_ref[...]

    return pl.pallas_call(
        body,
        grid=(n // MLP_ROWS,),
        in_specs=[
            pl.BlockSpec((MLP_ROWS, 4 * EMBED), lambda i: (i, 0)),
            pl.BlockSpec((4 * EMBED, EMBED), lambda i: (0, 0)),
            pl.BlockSpec((1, EMBED), lambda i: (0, 0)),
            pl.BlockSpec((EMBED, EMBED), lambda i: (0, 0)),
            pl.BlockSpec((1, EMBED), lambda i: (0, 0)),
        ],
        out_specs=pl.BlockSpec((MLP_ROWS, EMBED), lambda i: (i, 0)),
        out_shape=jax.ShapeDtypeStruct((n, EMBED), jnp.float32),
    )(emb, W1, b1, W2, b2)


def kernel(x, table, W1, b1, W2, b2):
    bsz, seq, c = x.shape
    n = bsz * seq
    idx = x.reshape(1, n * c).astype(jnp.int32)
    rows = _sc_gather(table, idx)
    emb = rows.reshape(n, c * EMBED)
    out = _tc_mlp(emb, W1, b1.reshape(1, -1), W2, b2.reshape(1, -1))
    return out.reshape(bsz, seq, EMBED)


# speedup vs baseline: 4.0022x; 1.1656x over previous
import functools

import jax
import jax.numpy as jnp
from jax import lax
from jax.experimental import pallas as pl
from jax.experimental.pallas import tpu as pltpu
from jax.experimental.pallas import tpu_sc as plsc

EMBED = 64
VOCAB = 1024
LANES = 16
TOK_PER_STEP = 128
IDX_PER_STEP = 4 * TOK_PER_STEP
CHUNK = 128
MLP_ROWS = 2048


def _tc_fold(table, W1):
    def body(t_ref, w1_ref, out_ref):
        out_ref[...] = jnp.dot(t_ref[...], w1_ref[...],
                               preferred_element_type=jnp.float32,
                               precision=lax.Precision.HIGHEST)

    return pl.pallas_call(
        body,
        grid=(4,),
        in_specs=[
            pl.BlockSpec((VOCAB, EMBED), lambda c: (0, 0)),
            pl.BlockSpec((EMBED, EMBED), lambda c: (c, 0)),
        ],
        out_specs=pl.BlockSpec((VOCAB, EMBED), lambda c: (c, 0)),
        out_shape=jax.ShapeDtypeStruct((4 * VOCAB, EMBED), jnp.float32),
    )(table, W1)


def _sc_gather_sum(tcat, idx):
    n_idx = idx.shape[0]
    n_tok = n_idx // 4
    mesh = plsc.VectorSubcoreMesh(core_axis_name="c", subcore_axis_name="s")

    @functools.partial(
        pl.kernel,
        out_type=jax.ShapeDtypeStruct((n_tok, EMBED), jnp.float32),
        mesh=mesh,
        scratch_types=[
            pltpu.VMEM((4, CHUNK), jnp.int32),
            pltpu.VMEM((4, CHUNK), jnp.int32),
            pltpu.VMEM((IDX_PER_STEP, EMBED), jnp.float32),
            pltpu.VMEM((TOK_PER_STEP, EMBED), jnp.float32),
            pltpu.VMEM_SHARED((16 * TOK_PER_STEP, EMBED), jnp.float32),
        ],
        compiler_params=pltpu.CompilerParams(use_tc_tiling_on_sc=False),
    )
    def gather_kernel(tcat_hbm, idx_hbm, out_hbm, gidx, tokidx, rows, zeros,
                      acc_shared):
        sid = lax.axis_index("s")
        acc_base = sid * TOK_PER_STEP

        @pl.loop(0, TOK_PER_STEP)
        def _(r):
            @pl.loop(0, EMBED // LANES)
            def _(k):
                zeros[pl.ds(r, 1), pl.ds(k * LANES, LANES)] = (
                    jnp.zeros((1, LANES), jnp.float32))

        @pl.loop(0, 4)
        def _(j):
            @pl.loop(0, CHUNK // LANES)
            def _(k):
                lane = lax.iota(jnp.int32, LANES) + (j * CHUNK + k * LANES)
                tid = lax.shift_right_logical(lane, 2) + acc_base
                tokidx[pl.ds(j, 1), pl.ds(k * LANES, LANES)] = (
                    tid.reshape(1, LANES))

        def body(i_vmem, o_vmem):
            @pl.loop(0, 4)
            def _(j):
                @pl.loop(0, CHUNK // LANES)
                def _(k):
                    raw = i_vmem[pl.ds(j * CHUNK + k * LANES, LANES)]
                    off = lax.shift_left(
                        lax.iota(jnp.int32, LANES) & 3, 10)
                    gidx[pl.ds(j, 1), pl.ds(k * LANES, LANES)] = (
                        (raw + off).reshape(1, LANES))

            pltpu.sync_copy(zeros, acc_shared.at[pl.ds(acc_base, TOK_PER_STEP)])
            for j in range(4):
                pltpu.sync_copy(tcat_hbm.at[gidx.at[j]],
                                rows.at[pl.ds(j * CHUNK, CHUNK)])
            for j in range(4):
                pltpu.sync_copy(rows.at[pl.ds(j * CHUNK, CHUNK)],
                                acc_shared.at[tokidx.at[j]], add=True)
            pltpu.sync_copy(acc_shared.at[pl.ds(acc_base, TOK_PER_STEP)], o_vmem)

        pltpu.emit_pipeline(
            body,
            grid=(n_idx // IDX_PER_STEP,),
            in_specs=[pl.BlockSpec((IDX_PER_STEP,), lambda i: (i,))],
            out_specs=[pl.BlockSpec((TOK_PER_STEP, EMBED), lambda i: (i, 0))],
            core_axis_name=("c", "s"),
            dimension_semantics=(pltpu.PARALLEL,),
        )(idx_hbm, out_hbm)

    return gather_kernel(tcat, idx)


def _tc_mlp(h_pre, b1, W2, b2):
    n = h_pre.shape[0]

    def body(h_ref, b1_ref, w2_ref, b2_ref, out_ref):
        h = h_ref[...] + b1_ref[...]
        h = h * jax.nn.sigmoid(h)
        out_ref[...] = jnp.dot(h, w2_ref[...],
                               preferred_element_type=jnp.float32,
                               precision=lax.Precision.HIGHEST) + b2_ref[...]

    return pl.pallas_call(
        body,
        grid=(n // MLP_ROWS,),
        in_specs=[
            pl.BlockSpec((MLP_ROWS, EMBED), lambda i: (i, 0)),
            pl.BlockSpec((1, EMBED), lambda i: (0, 0)),
            pl.BlockSpec((EMBED, EMBED), lambda i: (0, 0)),
            pl.BlockSpec((1, EMBED), lambda i: (0, 0)),
        ],
        out_specs=pl.BlockSpec((MLP_ROWS, EMBED), lambda i: (i, 0)),
        out_shape=jax.ShapeDtypeStruct((n, EMBED), jnp.float32),
    )(h_pre, b1, W2, b2)


def kernel(x, table, W1, b1, W2, b2):
    bsz, seq, c = x.shape
    n = bsz * seq
    idx = x.reshape(n * c).astype(jnp.int32)
    tcat = _tc_fold(table, W1)
    h_pre = _sc_gather_sum(tcat, idx)
    out = _tc_mlp(h_pre, b1.reshape(1, -1), W2, b2.reshape(1, -1))
    return out.reshape(bsz, seq, EMBED)


# speedup vs baseline: 4.1789x; 1.0442x over previous
import functools

import jax
import jax.numpy as jnp
from jax import lax
from jax.experimental import pallas as pl
from jax.experimental.pallas import tpu as pltpu
from jax.experimental.pallas import tpu_sc as plsc

EMBED = 64
VOCAB = 1024
LANES = 16
TOK_PER_STEP = 128
IDX_PER_STEP = 4 * TOK_PER_STEP
CHUNK = 128
MLP_ROWS = 2048


def _tc_fold(table, W1):
    def body(t_ref, w1_ref, out_ref):
        out_ref[...] = jnp.dot(t_ref[...], w1_ref[...],
                               preferred_element_type=jnp.float32,
                               precision=lax.Precision.HIGHEST)

    return pl.pallas_call(
        body,
        grid=(4,),
        in_specs=[
            pl.BlockSpec((VOCAB, EMBED), lambda c: (0, 0)),
            pl.BlockSpec((EMBED, EMBED), lambda c: (c, 0)),
        ],
        out_specs=pl.BlockSpec((VOCAB, EMBED), lambda c: (c, 0)),
        out_shape=jax.ShapeDtypeStruct((4 * VOCAB, EMBED), jnp.float32),
    )(table, W1)


def _sc_gather_sum(tcat, idx):
    n_idx = idx.shape[1]
    n_tok = n_idx // 4
    mesh = plsc.VectorSubcoreMesh(core_axis_name="c", subcore_axis_name="s")

    @functools.partial(
        pl.kernel,
        out_type=jax.ShapeDtypeStruct((n_tok, EMBED), jnp.float32),
        mesh=mesh,
        scratch_types=[
            pltpu.VMEM((4, CHUNK), jnp.int32),
            pltpu.VMEM((4, CHUNK), jnp.int32),
            pltpu.VMEM((IDX_PER_STEP, EMBED), jnp.float32),
            pltpu.VMEM((TOK_PER_STEP, EMBED), jnp.float32),
            pltpu.VMEM_SHARED((16 * TOK_PER_STEP, EMBED), jnp.float32),
        ],
        compiler_params=pltpu.CompilerParams(use_tc_tiling_on_sc=False),
    )
    def gather_kernel(tcat_hbm, idx_hbm, out_hbm, gidx, tokidx, rows, zeros,
                      acc_shared):
        sid = lax.axis_index("s")
        acc_base = sid * TOK_PER_STEP

        @pl.loop(0, TOK_PER_STEP)
        def _(r):
            @pl.loop(0, EMBED // LANES)
            def _(k):
                zeros[pl.ds(r, 1), pl.ds(k * LANES, LANES)] = (
                    jnp.zeros((1, LANES), jnp.float32))

        @pl.loop(0, 4)
        def _(j):
            @pl.loop(0, CHUNK // LANES)
            def _(k):
                lane = lax.iota(jnp.int32, LANES) + (j * CHUNK + k * LANES)
                tid = lax.shift_right_logical(lane, 2) + acc_base
                tokidx[pl.ds(j, 1), pl.ds(k * LANES, LANES)] = (
                    tid.reshape(1, LANES))

        def body(i_vmem, o_vmem):
            @pl.loop(0, 4)
            def _(j):
                @pl.loop(0, CHUNK // LANES)
                def _(k):
                    raw = i_vmem[pl.ds(0, 1), pl.ds(j * CHUNK + k * LANES, LANES)]
                    off = lax.shift_left(
                        lax.iota(jnp.int32, LANES) & 3, 10).reshape(1, LANES)
                    gidx[pl.ds(j, 1), pl.ds(k * LANES, LANES)] = raw + off

            pltpu.sync_copy(zeros, acc_shared.at[pl.ds(acc_base, TOK_PER_STEP)])
            for j in range(4):
                pltpu.sync_copy(tcat_hbm.at[gidx.at[j]],
                                rows.at[pl.ds(j * CHUNK, CHUNK)])
            for j in range(4):
                pltpu.sync_copy(rows.at[pl.ds(j * CHUNK, CHUNK)],
                                acc_shared.at[tokidx.at[j]], add=True)
            pltpu.sync_copy(acc_shared.at[pl.ds(acc_base, TOK_PER_STEP)], o_vmem)

        pltpu.emit_pipeline(
            body,
            grid=(n_idx // IDX_PER_STEP,),
            in_specs=[pl.BlockSpec((1, IDX_PER_STEP), lambda i: (0, i))],
            out_specs=[pl.BlockSpec((TOK_PER_STEP, EMBED), lambda i: (i, 0))],
            core_axis_name=("c", "s"),
            dimension_semantics=(pltpu.PARALLEL,),
        )(idx_hbm, out_hbm)

    return gather_kernel(tcat, idx)


def _tc_mlp(h_pre, b1, W2, b2):
    n = h_pre.shape[0]

    def body(h_ref, b1_ref, w2_ref, b2_ref, out_ref):
        h = h_ref[...] + b1_ref[...]
        h = h * jax.nn.sigmoid(h)
        out_ref[...] = jnp.dot(h, w2_ref[...],
                               preferred_element_type=jnp.float32) + b2_ref[...]

    return pl.pallas_call(
        body,
        grid=(n // MLP_ROWS,),
        in_specs=[
            pl.BlockSpec((MLP_ROWS, EMBED), lambda i: (i, 0)),
            pl.BlockSpec((1, EMBED), lambda i: (0, 0)),
            pl.BlockSpec((EMBED, EMBED), lambda i: (0, 0)),
            pl.BlockSpec((1, EMBED), lambda i: (0, 0)),
        ],
        out_specs=pl.BlockSpec((MLP_ROWS, EMBED), lambda i: (i, 0)),
        out_shape=jax.ShapeDtypeStruct((n, EMBED), jnp.float32),
    )(h_pre, b1, W2, b2)


def kernel(x, table, W1, b1, W2, b2):
    bsz, seq, c = x.shape
    n = bsz * seq
    idx = x.reshape(1, n * c).astype(jnp.int32)
    tcat = _tc_fold(table, W1)
    h_pre = _sc_gather_sum(tcat, idx)
    out = _tc_mlp(h_pre, b1.reshape(1, -1), W2, b2.reshape(1, -1))
    return out.reshape(bsz, seq, EMBED)


# speedup vs baseline: 5.2300x; 1.2515x over previous
import functools

import jax
import jax.numpy as jnp
from jax import lax
from jax.experimental import pallas as pl
from jax.experimental.pallas import tpu as pltpu
from jax.experimental.pallas import tpu_sc as plsc

EMBED = 64
VOCAB = 1024
LANES = 16
CHUNK = 128
TOK_PER_STEP = 128
IDX_PER_STEP = 4 * CHUNK
MLP_ROWS = 512


def _tc_fold(table, W1):
    def body(t_ref, w1_ref, out_ref):
        out_ref[...] = jnp.dot(t_ref[...], w1_ref[...],
                               preferred_element_type=jnp.float32,
                               precision=lax.Precision.HIGHEST)

    return pl.pallas_call(
        body,
        grid=(4,),
        in_specs=[
            pl.BlockSpec((VOCAB, EMBED), lambda c: (0, 0)),
            pl.BlockSpec((EMBED, EMBED), lambda c: (c, 0)),
        ],
        out_specs=pl.BlockSpec((VOCAB, EMBED), lambda c: (c, 0)),
        out_shape=jax.ShapeDtypeStruct((4 * VOCAB, EMBED), jnp.float32),
    )(table, W1)


def _sc_gather_sum(tcat, idx):
    n_rows = idx.shape[0]
    n_tok = n_rows * TOK_PER_STEP
    mesh = plsc.VectorSubcoreMesh(core_axis_name="c", subcore_axis_name="s")

    @functools.partial(
        pl.kernel,
        out_type=jax.ShapeDtypeStruct((n_tok, EMBED), jnp.float32),
        mesh=mesh,
        scratch_types=[
            pltpu.VMEM((4, CHUNK), jnp.int32),
            pltpu.VMEM((1, CHUNK), jnp.int32),
            pltpu.VMEM((IDX_PER_STEP, EMBED), jnp.float32),
            pltpu.VMEM((TOK_PER_STEP, EMBED), jnp.float32),
            pltpu.VMEM_SHARED((16 * TOK_PER_STEP, EMBED), jnp.float32),
        ],
        compiler_params=pltpu.CompilerParams(use_tc_tiling_on_sc=False),
    )
    def gather_kernel(tcat_hbm, idx_hbm, out_hbm, gidx, tokidx, rows, zeros,
                      acc_shared):
        sid = lax.axis_index("s")
        acc_base = sid * TOK_PER_STEP

        @pl.loop(0, TOK_PER_STEP)
        def _(r):
            @pl.loop(0, EMBED // LANES)
            def _(k):
                zeros[pl.ds(r, 1), pl.ds(k * LANES, LANES)] = (
                    jnp.zeros((1, LANES), jnp.float32))

        @pl.loop(0, CHUNK // LANES)
        def _(k):
            tid = lax.iota(jnp.int32, LANES) + (k * LANES + acc_base)
            tokidx[pl.ds(0, 1), pl.ds(k * LANES, LANES)] = tid.reshape(1, LANES)

        def body(i_vmem, o_vmem):
            for j in range(4):
                @pl.loop(0, CHUNK // LANES)
                def _(k, j=j):
                    raw = i_vmem[pl.ds(0, 1), pl.ds(j * CHUNK + k * LANES, LANES)]
                    gidx[pl.ds(j, 1), pl.ds(k * LANES, LANES)] = raw + (j << 10)

            pltpu.sync_copy(zeros, acc_shared.at[pl.ds(acc_base, TOK_PER_STEP)])
            for j in range(4):
                pltpu.sync_copy(tcat_hbm.at[gidx.at[j]],
                                rows.at[pl.ds(j * CHUNK, CHUNK)])
            for j in range(4):
                pltpu.sync_copy(rows.at[pl.ds(j * CHUNK, CHUNK)],
                                acc_shared.at[tokidx.at[0]], add=True)
            pltpu.sync_copy(acc_shared.at[pl.ds(acc_base, TOK_PER_STEP)], o_vmem)

        pltpu.emit_pipeline(
            body,
            grid=(n_rows,),
            in_specs=[pl.BlockSpec((1, IDX_PER_STEP), lambda i: (i, 0))],
            out_specs=[pl.BlockSpec((TOK_PER_STEP, EMBED), lambda i: (i, 0))],
            core_axis_name=("c", "s"),
            dimension_semantics=(pltpu.PARALLEL,),
        )(idx_hbm, out_hbm)

    return gather_kernel(tcat, idx)


def _tc_mlp_t(h_pre, b1, W2t, b2, n_seq, n_batch):
    blocks_per_seq = n_batch // MLP_ROWS

    def body(h_ref, b1_ref, w2t_ref, b2_ref, out_ref):
        h = h_ref[...] + b1_ref[...]
        h = h * jax.nn.sigmoid(h)
        ht = lax.dot_general(w2t_ref[...], h,
                             dimension_numbers=(((1,), (1,)), ((), ())),
                             preferred_element_type=jnp.float32)
        out_ref[...] = (ht + b2_ref[...])[None]

    return pl.pallas_call(
        body,
        grid=(n_seq, blocks_per_seq),
        in_specs=[
            pl.BlockSpec((MLP_ROWS, EMBED),
                         lambda s, i: (s * blocks_per_seq + i, 0)),
            pl.BlockSpec((1, EMBED), lambda s, i: (0, 0)),
            pl.BlockSpec((EMBED, EMBED), lambda s, i: (0, 0)),
            pl.BlockSpec((EMBED, 1), lambda s, i: (0, 0)),
        ],
        out_specs=pl.BlockSpec((1, EMBED, MLP_ROWS), lambda s, i: (s, 0, i)),
        out_shape=jax.ShapeDtypeStruct((n_seq, EMBED, n_batch), jnp.float32),
    )(h_pre, b1, W2t, b2)


def kernel(x, table, W1, b1, W2, b2):
    bsz, seq, c = x.shape
    idx = (x.astype(jnp.int32)
           .reshape(bsz // CHUNK, CHUNK, seq, c)
           .transpose(2, 0, 3, 1)
           .reshape(seq * (bsz // CHUNK), c * CHUNK))
    tcat = _tc_fold(table, W1)
    h_pre = _sc_gather_sum(tcat, idx)
    out_t = _tc_mlp_t(h_pre, b1.reshape(1, -1), W2.T, b2.reshape(-1, 1),
                      seq, bsz)
    return out_t.transpose(2, 0, 1)


# speedup vs baseline: 7.2688x; 1.3898x over previous
import functools

import jax
import jax.numpy as jnp
from jax import lax
from jax.experimental import pallas as pl
from jax.experimental.pallas import tpu as pltpu
from jax.experimental.pallas import tpu_sc as plsc

EMBED = 64
VOCAB = 1024
LANES = 16
CHUNK = 128
TOK_PER_STEP = 128
IDX_PER_STEP = 4 * CHUNK
MLP_ROWS = 8192


def _tc_fold(table, W1):
    def body(t_ref, w1_ref, out_ref):
        out_ref[...] = jnp.dot(t_ref[...], w1_ref[...],
                               preferred_element_type=jnp.float32,
                               precision=lax.Precision.HIGHEST)

    return pl.pallas_call(
        body,
        grid=(4,),
        in_specs=[
            pl.BlockSpec((VOCAB, EMBED), lambda c: (0, 0)),
            pl.BlockSpec((EMBED, EMBED), lambda c: (c, 0)),
        ],
        out_specs=pl.BlockSpec((VOCAB, EMBED), lambda c: (c, 0)),
        out_shape=jax.ShapeDtypeStruct((4 * VOCAB, EMBED), jnp.float32),
    )(table, W1)


def _sc_gather_sum(tcat, idx):
    n_rows = idx.shape[0]
    n_tok = n_rows * TOK_PER_STEP
    mesh = plsc.VectorSubcoreMesh(core_axis_name="c", subcore_axis_name="s")

    @functools.partial(
        pl.kernel,
        out_type=jax.ShapeDtypeStruct((n_tok, EMBED), jnp.float32),
        mesh=mesh,
        scratch_types=[
            pltpu.VMEM((4, CHUNK), jnp.int32),
            pltpu.VMEM((1, CHUNK), jnp.int32),
            pltpu.VMEM((IDX_PER_STEP, EMBED), jnp.float32),
            pltpu.VMEM((TOK_PER_STEP, EMBED), jnp.float32),
            pltpu.VMEM_SHARED((16 * TOK_PER_STEP, EMBED), jnp.float32),
        ],
        compiler_params=pltpu.CompilerParams(use_tc_tiling_on_sc=False),
    )
    def gather_kernel(tcat_hbm, idx_hbm, out_hbm, gidx, tokidx, rows, zeros,
                      acc_shared):
        sid = lax.axis_index("s")
        acc_base = sid * TOK_PER_STEP

        @pl.loop(0, TOK_PER_STEP)
        def _(r):
            @pl.loop(0, EMBED // LANES)
            def _(k):
                zeros[pl.ds(r, 1), pl.ds(k * LANES, LANES)] = (
                    jnp.zeros((1, LANES), jnp.float32))

        @pl.loop(0, CHUNK // LANES)
        def _(k):
            tid = lax.iota(jnp.int32, LANES) + (k * LANES + acc_base)
            tokidx[pl.ds(0, 1), pl.ds(k * LANES, LANES)] = tid.reshape(1, LANES)

        def body(i_vmem, o_vmem):
            for j in range(4):
                @pl.loop(0, CHUNK // LANES)
                def _(k, j=j):
                    raw = i_vmem[pl.ds(0, 1), pl.ds(j * CHUNK + k * LANES, LANES)]
                    gidx[pl.ds(j, 1), pl.ds(k * LANES, LANES)] = raw + (j << 10)

            pltpu.sync_copy(zeros, acc_shared.at[pl.ds(acc_base, TOK_PER_STEP)])
            for j in range(4):
                pltpu.sync_copy(tcat_hbm.at[gidx.at[j]],
                                rows.at[pl.ds(j * CHUNK, CHUNK)])
            for j in range(4):
                pltpu.sync_copy(rows.at[pl.ds(j * CHUNK, CHUNK)],
                                acc_shared.at[tokidx.at[0]], add=True)
            pltpu.sync_copy(acc_shared.at[pl.ds(acc_base, TOK_PER_STEP)], o_vmem)

        pltpu.emit_pipeline(
            body,
            grid=(n_rows,),
            in_specs=[pl.BlockSpec((1, IDX_PER_STEP), lambda i: (i, 0))],
            out_specs=[pl.BlockSpec((TOK_PER_STEP, EMBED), lambda i: (i, 0))],
            core_axis_name=("c", "s"),
            dimension_semantics=(pltpu.PARALLEL,),
        )(idx_hbm, out_hbm)

    return gather_kernel(tcat, idx)


def _tc_mlp_paired(h2, b1p, W2d, b2p):
    n2 = h2.shape[0]

    def body(h_ref, b1_ref, w2_ref, b2_ref, out_ref):
        h = h_ref[...] + b1_ref[...]
        h = h * jax.nn.sigmoid(h)
        out_ref[...] = jnp.dot(h, w2_ref[...],
                               preferred_element_type=jnp.float32) + b2_ref[...]

    return pl.pallas_call(
        body,
        grid=(n2 // MLP_ROWS,),
        in_specs=[
            pl.BlockSpec((MLP_ROWS, 2 * EMBED), lambda i: (i, 0)),
            pl.BlockSpec((1, 2 * EMBED), lambda i: (0, 0)),
            pl.BlockSpec((2 * EMBED, 2 * EMBED), lambda i: (0, 0)),
            pl.BlockSpec((1, 2 * EMBED), lambda i: (0, 0)),
        ],
        out_specs=pl.BlockSpec((MLP_ROWS, 2 * EMBED), lambda i: (i, 0)),
        out_shape=jax.ShapeDtypeStruct((n2, 2 * EMBED), jnp.float32),
    )(h2, b1p, W2d, b2p)


def kernel(x, table, W1, b1, W2, b2):
    bsz, seq, c = x.shape
    n = bsz * seq
    idx = (x.astype(jnp.int32)
           .reshape(bsz // CHUNK, CHUNK, seq, c)
           .transpose(2, 0, 3, 1)
           .reshape(seq * (bsz // CHUNK), c * CHUNK))
    tcat = _tc_fold(table, W1)
    h_pre = _sc_gather_sum(tcat, idx)
    h2 = h_pre.reshape(n // 2, 2 * EMBED)
    z = jnp.zeros((EMBED, EMBED), jnp.float32)
    W2d = jnp.block([[W2, z], [z, W2]])
    b1p = jnp.tile(b1, 2).reshape(1, -1)
    b2p = jnp.tile(b2, 2).reshape(1, -1)
    out2 = _tc_mlp_paired(h2, b1p, W2d, b2p)
    return out2.reshape(seq, bsz, EMBED).transpose(1, 0, 2)
